# Initial kernel scaffold; baseline (speedup 1.0000x reference)
import jax
import jax.numpy as jnp
from jax.experimental import pallas as pl


def kernel(node_feats, edge_feats, edge_index, W_e, b_e, eps, W1, b1, gamma, beta, W2, b2):
    raise NotImplementedError("write your pallas kernel here")


# speedup vs baseline: 1.8310x; 1.8310x over previous
import jax
import jax.numpy as jnp
from jax import lax
from jax.experimental import pallas as pl
from jax.experimental.pallas import tpu as pltpu
from jax.experimental.pallas import tpu_sc as plsc

N = 10000
E = 160000
D = 256
DH = 128
DE = 16
EP = 163840
NT = 16
NC = 2
CH = 128
EPT = EP // NT
NCHUNK = EPT // CH
RPT = 640
ACC_ROWS = NT * RPT
DUMMY = N


def _eproj_body(ef_ref, we_ref, be_ref, out_ref):
    z = jnp.dot(ef_ref[...], we_ref[...], preferred_element_type=jnp.float32)
    z = z + be_ref[...]
    out_ref[0] = z[:, :DH]
    out_ref[1] = z[:, DH:]


def _eproj(ef_pad, W_e, b_e):
    BE = 4096
    return pl.pallas_call(
        _eproj_body,
        grid=(EP // BE,),
        in_specs=[
            pl.BlockSpec((BE, DE), lambda i: (i, 0)),
            pl.BlockSpec((DE, D), lambda i: (0, 0)),
            pl.BlockSpec((1, D), lambda i: (0, 0)),
        ],
        out_specs=pl.BlockSpec((NC, BE, DH), lambda i: (0, i, 0)),
        out_shape=jax.ShapeDtypeStruct((NC, EP, DH), jnp.float32),
    )(ef_pad, W_e, b_e.reshape(1, D))


def _sc_agg(node_cat, ep_flat, src_adj2, dst2):
    mesh = plsc.VectorSubcoreMesh(core_axis_name="c", subcore_axis_name="s")

    def body(node_hbm, ep_hbm, srcadj_hbm, dst_hbm, out_hbm,
             src_buf, dst_buf, gat_buf, acc):
        c = lax.axis_index("c")
        s = lax.axis_index("s")

        def zero_row(i, _):
            for v in range(DH // 16):
                gat_buf[i, pl.ds(v * 16, 16)] = jnp.zeros((16,), jnp.float32)
            return 0
        lax.fori_loop(0, CH, zero_row, 0)
        for k in range(RPT // CH):
            pltpu.sync_copy(gat_buf, acc.at[pl.ds(s * RPT + k * CH, CH)])
        plsc.subcore_barrier()

        crow = c * (EP // CH) + s * NCHUNK
        pltpu.sync_copy(srcadj_hbm.at[pl.ds(crow, NCHUNK)], src_buf)
        pltpu.sync_copy(dst_hbm.at[pl.ds(s * NCHUNK, NCHUNK)], dst_buf)

        def chunk(j, _):
            erow = c * EP + s * EPT + j * CH
            pltpu.sync_copy(ep_hbm.at[pl.ds(erow, CH)], gat_buf)
            pltpu.sync_copy(node_hbm.at[src_buf.at[j]], gat_buf, add=True)

            def relu_row(i, _):
                for v in range(DH // 16):
                    sl = pl.ds(v * 16, 16)
                    gat_buf[i, sl] = jnp.maximum(gat_buf[i, sl], 0.0)
                return 0
            lax.fori_loop(0, CH, relu_row, 0)
            pltpu.sync_copy(gat_buf, acc.at[dst_buf.at[j]], add=True)
            return 0
        lax.fori_loop(0, NCHUNK, chunk, 0)
        plsc.subcore_barrier()

        for k in range(RPT // CH):
            r = s * RPT + k * CH
            pltpu.sync_copy(acc.at[pl.ds(r, CH)],
                            out_hbm.at[pl.ds(c * ACC_ROWS + r, CH)])

    f = pl.kernel(
        body,
        out_type=jax.ShapeDtypeStruct((NC * ACC_ROWS, DH), jnp.float32),
        mesh=mesh,
        scratch_types=[
            pltpu.VMEM((NCHUNK, CH), jnp.int32),
            pltpu.VMEM((NCHUNK, CH), jnp.int32),
            pltpu.VMEM((CH, DH), jnp.float32),
            pltpu.VMEM_SHARED((ACC_ROWS, DH), jnp.float32),
        ],
    )
    return f(node_cat, ep_flat, src_adj2, dst2)


def _mlp1_body(eps_ref, alo_ref, ahi_ref, nf_ref, w1_ref, b1_ref,
               z_ref, sum_ref, sq_ref):
    i = pl.program_id(0)
    scale = 1.0 + eps_ref[0]
    rl = alo_ref[...] + scale * nf_ref[:, :DH]
    rh = ahi_ref[...] + scale * nf_ref[:, DH:]
    z = jnp.dot(rl, w1_ref[:DH, :], preferred_element_type=jnp.float32)
    z = z + jnp.dot(rh, w1_ref[DH:, :], preferred_element_type=jnp.float32)
    z = z + b1_ref[...]
    z_ref[...] = z

    @pl.when(i == 0)
    def _():
        sum_ref[...] = jnp.zeros_like(sum_ref)
        sq_ref[...] = jnp.zeros_like(sq_ref)
    sum_ref[...] += jnp.sum(z, axis=0, keepdims=True)
    sq_ref[...] += jnp.sum(z * z, axis=0, keepdims=True)


def _mlp1(eps, agg_lo, agg_hi, node_feats, W1, b1):
    RB = 400
    return pl.pallas_call(
        _mlp1_body,
        grid=(N // RB,),
        in_specs=[
            pl.BlockSpec(memory_space=pltpu.SMEM),
            pl.BlockSpec((RB, DH), lambda i: (i, 0)),
            pl.BlockSpec((RB, DH), lambda i: (i, 0)),
            pl.BlockSpec((RB, D), lambda i: (i, 0)),
            pl.BlockSpec((D, 2 * D), lambda i: (0, 0)),
            pl.BlockSpec((1, 2 * D), lambda i: (0, 0)),
        ],
        out_specs=[
            pl.BlockSpec((RB, 2 * D), lambda i: (i, 0)),
            pl.BlockSpec((1, 2 * D), lambda i: (0, 0)),
            pl.BlockSpec((1, 2 * D), lambda i: (0, 0)),
        ],
        out_shape=[
            jax.ShapeDtypeStruct((N, 2 * D), jnp.float32),
            jax.ShapeDtypeStruct((1, 2 * D), jnp.float32),
            jax.ShapeDtypeStruct((1, 2 * D), jnp.float32),
        ],
    )(eps, agg_lo, agg_hi, node_feats, W1, b1.reshape(1, 2 * D))


def _mlp2_body(z_ref, sum_ref, sq_ref, g_ref, b_ref, w2_ref, b2_ref, out_ref):
    mean = sum_ref[...] * (1.0 / N)
    var = sq_ref[...] * (1.0 / N) - mean * mean
    inv = lax.rsqrt(var + 1e-5)
    sc = g_ref[...] * inv
    sh = b_ref[...] - mean * sc
    a = jnp.maximum(z_ref[...] * sc + sh, 0.0)
    out = jnp.dot(a, w2_ref[...], preferred_element_type=jnp.float32)
    out_ref[...] = out + b2_ref[...]


def _mlp2(z, sums, sqs, gamma, beta, W2, b2):
    RB = 400
    return pl.pallas_call(
        _mlp2_body,
        grid=(N // RB,),
        in_specs=[
            pl.BlockSpec((RB, 2 * D), lambda i: (i, 0)),
            pl.BlockSpec((1, 2 * D), lambda i: (0, 0)),
            pl.BlockSpec((1, 2 * D), lambda i: (0, 0)),
            pl.BlockSpec((1, 2 * D), lambda i: (0, 0)),
            pl.BlockSpec((1, 2 * D), lambda i: (0, 0)),
            pl.BlockSpec((2 * D, D), lambda i: (0, 0)),
            pl.BlockSpec((1, D), lambda i: (0, 0)),
        ],
        out_specs=pl.BlockSpec((RB, D), lambda i: (i, 0)),
        out_shape=jax.ShapeDtypeStruct((N, D), jnp.float32),
    )(z, sums, sqs, gamma.reshape(1, 2 * D), beta.reshape(1, 2 * D),
      W2, b2.reshape(1, D))


def kernel(node_feats, edge_feats, edge_index, W_e, b_e, eps, W1, b1,
           gamma, beta, W2, b2):
    pad = EP - E
    src = edge_index[0]
    dst = edge_index[1]
    src_p = jnp.concatenate([src, jnp.zeros((pad,), jnp.int32)])
    dst_p = jnp.concatenate([dst, jnp.full((pad,), DUMMY, jnp.int32)])
    src_adj2 = jnp.concatenate([src_p, src_p + N]).reshape(2 * EP // CH, CH)
    dst2 = dst_p.reshape(EP // CH, CH)
    ef_p = jnp.concatenate([edge_feats,
                            jnp.zeros((pad, DE), jnp.float32)])
    node_cat = jnp.concatenate([node_feats[:, :DH], node_feats[:, DH:]],
                               axis=0)

    ep2 = _eproj(ef_p, W_e, b_e)
    agg2 = _sc_agg(node_cat, ep2.reshape(NC * EP, DH), src_adj2, dst2)
    agg_lo = agg2[:N]
    agg_hi = agg2[ACC_ROWS:ACC_ROWS + N]
    z, sums, sqs = _mlp1(eps, agg_lo, agg_hi, node_feats, W1, b1)
    return _mlp2(z, sums, sqs, gamma, beta, W2, b2)


# speedup vs baseline: 2.2906x; 1.2510x over previous
import jax
import jax.numpy as jnp
from jax import lax
from jax.experimental import pallas as pl
from jax.experimental.pallas import tpu as pltpu
from jax.experimental.pallas import tpu_sc as plsc

N = 10000
E = 160000
D = 256
DH = 128
DE = 16
EP = 163840
NT = 16
NC = 2
CH = 128
EPT = EP // NT
NCHUNK = EPT // CH
RPT = 632
ACC_ROWS = NT * RPT
DUMMY = N


def _eproj_body(ef_ref, we_ref, be_ref, out_ref):
    z = jnp.dot(ef_ref[...], we_ref[...], preferred_element_type=jnp.float32)
    z = z + be_ref[...]
    out_ref[0] = z[:, :DH]
    out_ref[1] = z[:, DH:]


def _eproj(ef_pad, W_e, b_e):
    BE = 4096
    return pl.pallas_call(
        _eproj_body,
        grid=(EP // BE,),
        in_specs=[
            pl.BlockSpec((BE, DE), lambda i: (i, 0)),
            pl.BlockSpec((DE, D), lambda i: (0, 0)),
            pl.BlockSpec((1, D), lambda i: (0, 0)),
        ],
        out_specs=pl.BlockSpec((NC, BE, DH), lambda i: (0, i, 0)),
        out_shape=jax.ShapeDtypeStruct((NC, EP, DH), jnp.float32),
    )(ef_pad, W_e, b_e.reshape(1, D))


NBUF = 3


def _sc_agg(node_cat, ep_flat, src_adj, dst_p):
    mesh = plsc.VectorSubcoreMesh(core_axis_name="c", subcore_axis_name="s")

    def body(node_hbm, ep_hbm, srcadj_hbm, dst_hbm, out_hbm,
             sidx, didx, g0, g1, g2, acc,
             se0, se1, se2, sg0, sg1, sg2, ss0, ss1, ss2,
             si0, si1, si2, sd0, sd1, sd2):
        gats = (g0, g1, g2)
        se = (se0, se1, se2)
        sg = (sg0, sg1, sg2)
        ss = (ss0, ss1, ss2)
        si = (si0, si1, si2)
        sd = (sd0, sd1, sd2)
        c = lax.axis_index("c")
        s = lax.axis_index("s")

        def zero_row(i, _):
            for v in range(DH // 16):
                g0[i, pl.ds(v * 16, 16)] = jnp.zeros((16,), jnp.float32)
            return 0
        lax.fori_loop(0, CH, zero_row, 0)
        for k in range(RPT // CH):
            pltpu.sync_copy(g0, acc.at[pl.ds(s * RPT + k * CH, CH)])
        rem = RPT - (RPT // CH) * CH
        if rem:
            pltpu.sync_copy(g0.at[pl.ds(0, rem)],
                            acc.at[pl.ds(s * RPT + (RPT // CH) * CH, rem)])
        plsc.subcore_barrier()

        def ep_src(j):
            return ep_hbm.at[pl.ds(c * EP + s * EPT + j * CH, CH)]

        def sidx_src(j):
            return srcadj_hbm.at[pl.ds((c * EP + s * EPT + j * CH), CH)]

        def didx_src(j):
            return dst_hbm.at[pl.ds(s * EPT + j * CH, CH)]

        def group(g, _):
            for b in range(NBUF):
                j = g * NBUF + b
                jb, bb = j - 1, (b - 1) % NBUF
                jc, bc = j - 2, (b - 2) % NBUF

                @pl.when((j >= NBUF) & (j < NCHUNK))
                def _():
                    pltpu.make_async_copy(
                        gats[b], acc.at[didx.at[b]], ss[b]).wait()

                @pl.when(j < NCHUNK)
                def _():
                    pltpu.async_copy(sidx_src(j), sidx.at[b], si[b])
                    pltpu.async_copy(didx_src(j), didx.at[b], sd[b])
                    pltpu.async_copy(ep_src(j), gats[b], se[b])

                @pl.when((jb >= 0) & (jb < NCHUNK))
                def _():
                    pltpu.make_async_copy(ep_src(jb), gats[bb], se[bb]).wait()
                    pltpu.make_async_copy(
                        sidx_src(jb), sidx.at[bb], si[bb]).wait()
                    pltpu.async_copy(node_hbm.at[sidx.at[bb]], gats[bb],
                                     sg[bb], add=True)

                @pl.when((jc >= 0) & (jc < NCHUNK))
                def _():
                    pltpu.make_async_copy(
                        node_hbm.at[sidx.at[bc]], gats[bc], sg[bc]).wait()
                    pltpu.make_async_copy(
                        didx_src(jc), didx.at[bc], sd[bc]).wait()

                    def relu_row(i, _):
                        for r in range(2):
                            for v in range(DH // 16):
                                sl = pl.ds(v * 16, 16)
                                gats[bc][2 * i + r, sl] = jnp.maximum(
                                    gats[bc][2 * i + r, sl], 0.0)
                        return 0
                    lax.fori_loop(0, CH // 2, relu_row, 0)
                    pltpu.async_copy(gats[bc], acc.at[didx.at[bc]],
                                     ss[bc], add=True)
            return 0
        lax.fori_loop(0, (NCHUNK + 2) // NBUF + 1, group, 0)
        for b in range(NBUF):
            if any((j % NBUF) == b for j in range(NCHUNK - NBUF, NCHUNK)):
                pltpu.make_async_copy(
                    gats[b], acc.at[didx.at[b]], ss[b]).wait()
        plsc.subcore_barrier()

        for k in range(RPT // CH):
            r = s * RPT + k * CH
            pltpu.sync_copy(acc.at[pl.ds(r, CH)],
                            out_hbm.at[pl.ds(c * ACC_ROWS + r, CH)])
        if rem:
            r = s * RPT + (RPT // CH) * CH
            pltpu.sync_copy(acc.at[pl.ds(r, rem)],
                            out_hbm.at[pl.ds(c * ACC_ROWS + r, rem)])

    f = pl.kernel(
        body,
        out_type=jax.ShapeDtypeStruct((NC * ACC_ROWS, DH), jnp.float32),
        mesh=mesh,
        scratch_types=[
            pltpu.VMEM((NBUF, CH), jnp.int32),
            pltpu.VMEM((NBUF, CH), jnp.int32),
            pltpu.VMEM((CH, DH), jnp.float32),
            pltpu.VMEM((CH, DH), jnp.float32),
            pltpu.VMEM((CH, DH), jnp.float32),
            pltpu.VMEM_SHARED((ACC_ROWS, DH), jnp.float32),
        ] + [pltpu.SemaphoreType.DMA] * 15,
    )
    return f(node_cat, ep_flat, src_adj, dst_p)


def _mlp1_body(eps_ref, alo_ref, ahi_ref, nf_ref, w1_ref, b1_ref,
               z_ref, sum_ref, sq_ref):
    i = pl.program_id(0)
    scale = 1.0 + eps_ref[0]
    rl = alo_ref[...] + scale * nf_ref[:, :DH]
    rh = ahi_ref[...] + scale * nf_ref[:, DH:]
    z = jnp.dot(rl, w1_ref[:DH, :], preferred_element_type=jnp.float32)
    z = z + jnp.dot(rh, w1_ref[DH:, :], preferred_element_type=jnp.float32)
    z = z + b1_ref[...]
    z_ref[...] = z

    @pl.when(i == 0)
    def _():
        sum_ref[...] = jnp.zeros_like(sum_ref)
        sq_ref[...] = jnp.zeros_like(sq_ref)
    sum_ref[...] += jnp.sum(z, axis=0, keepdims=True)
    sq_ref[...] += jnp.sum(z * z, axis=0, keepdims=True)


def _mlp1(eps, agg_lo, agg_hi, node_feats, W1, b1):
    RB = 400
    return pl.pallas_call(
        _mlp1_body,
        grid=(N // RB,),
        in_specs=[
            pl.BlockSpec(memory_space=pltpu.SMEM),
            pl.BlockSpec((RB, DH), lambda i: (i, 0)),
            pl.BlockSpec((RB, DH), lambda i: (i, 0)),
            pl.BlockSpec((RB, D), lambda i: (i, 0)),
            pl.BlockSpec((D, 2 * D), lambda i: (0, 0)),
            pl.BlockSpec((1, 2 * D), lambda i: (0, 0)),
        ],
        out_specs=[
            pl.BlockSpec((RB, 2 * D), lambda i: (i, 0)),
            pl.BlockSpec((1, 2 * D), lambda i: (0, 0)),
            pl.BlockSpec((1, 2 * D), lambda i: (0, 0)),
        ],
        out_shape=[
            jax.ShapeDtypeStruct((N, 2 * D), jnp.float32),
            jax.ShapeDtypeStruct((1, 2 * D), jnp.float32),
            jax.ShapeDtypeStruct((1, 2 * D), jnp.float32),
        ],
    )(eps, agg_lo, agg_hi, node_feats, W1, b1.reshape(1, 2 * D))


def _mlp2_body(z_ref, sum_ref, sq_ref, g_ref, b_ref, w2_ref, b2_ref, out_ref):
    mean = sum_ref[...] * (1.0 / N)
    var = sq_ref[...] * (1.0 / N) - mean * mean
    inv = lax.rsqrt(var + 1e-5)
    sc = g_ref[...] * inv
    sh = b_ref[...] - mean * sc
    a = jnp.maximum(z_ref[...] * sc + sh, 0.0)
    out = jnp.dot(a, w2_ref[...], preferred_element_type=jnp.float32)
    out_ref[...] = out + b2_ref[...]


def _mlp2(z, sums, sqs, gamma, beta, W2, b2):
    RB = 400
    return pl.pallas_call(
        _mlp2_body,
        grid=(N // RB,),
        in_specs=[
            pl.BlockSpec((RB, 2 * D), lambda i: (i, 0)),
            pl.BlockSpec((1, 2 * D), lambda i: (0, 0)),
            pl.BlockSpec((1, 2 * D), lambda i: (0, 0)),
            pl.BlockSpec((1, 2 * D), lambda i: (0, 0)),
            pl.BlockSpec((1, 2 * D), lambda i: (0, 0)),
            pl.BlockSpec((2 * D, D), lambda i: (0, 0)),
            pl.BlockSpec((1, D), lambda i: (0, 0)),
        ],
        out_specs=pl.BlockSpec((RB, D), lambda i: (i, 0)),
        out_shape=jax.ShapeDtypeStruct((N, D), jnp.float32),
    )(z, sums, sqs, gamma.reshape(1, 2 * D), beta.reshape(1, 2 * D),
      W2, b2.reshape(1, D))


def kernel(node_feats, edge_feats, edge_index, W_e, b_e, eps, W1, b1,
           gamma, beta, W2, b2):
    pad = EP - E
    src = edge_index[0]
    dst = edge_index[1]
    src_p = jnp.concatenate([src, jnp.zeros((pad,), jnp.int32)])
    dst_p = jnp.concatenate([dst, jnp.full((pad,), DUMMY, jnp.int32)])
    src_adj = jnp.concatenate([src_p, src_p + N])
    ef_p = jnp.concatenate([edge_feats,
                            jnp.zeros((pad, DE), jnp.float32)])
    node_cat = jnp.concatenate([node_feats[:, :DH], node_feats[:, DH:]],
                               axis=0)

    ep2 = _eproj(ef_p, W_e, b_e)
    agg2 = _sc_agg(node_cat, ep2.reshape(NC * EP, DH), src_adj, dst_p)
    agg_lo = agg2[:N]
    agg_hi = agg2[ACC_ROWS:ACC_ROWS + N]
    z, sums, sqs = _mlp1(eps, agg_lo, agg_hi, node_feats, W1, b1)
    return _mlp2(z, sums, sqs, gamma, beta, W2, b2)
